# baseline (speedup 1.0000x reference)
import functools

import jax
import jax.numpy as jnp
from jax import lax
from jax.experimental import pallas as pl
from jax.experimental.pallas import tpu as pltpu
from jax.experimental.pallas import tpu_sc as plsc

_VOCAB = 100000
_HIDDEN = 32
_BATCH = 1024

_info = plsc.get_sparse_core_info()
_NC, _NS = _info.num_cores, _info.num_subcores
_NW = _NC * _NS
_B_PER_W = _BATCH // _NW

_sc_mesh = plsc.VectorSubcoreMesh(core_axis_name="c", subcore_axis_name="s")


@functools.partial(
    pl.kernel,
    mesh=_sc_mesh,
    out_type=jax.ShapeDtypeStruct((_HIDDEN, _BATCH), jnp.float32),
    scratch_types=[
        pltpu.VMEM((_BATCH,), jnp.int32),
        pltpu.VMEM((_VOCAB,), jnp.float32),
        pltpu.VMEM((_BATCH,), jnp.float32),
    ],
    compiler_params=pltpu.CompilerParams(
        use_tc_tiling_on_sc=True, needs_layout_passes=False),
)
def _sc_gather_t(idx_hbm, table_t_hbm, out_hbm, idx_v, row_v, g_v):
    wid = lax.axis_index("s") * _NC + lax.axis_index("c")
    pltpu.sync_copy(idx_hbm, idx_v)
    pltpu.sync_copy(table_t_hbm.at[wid], row_v)
    for k in range(_BATCH // 16):
        idx16 = idx_v[pl.ds(k * 16, 16)]
        g_v[pl.ds(k * 16, 16)] = plsc.load_gather(row_v, [idx16])
    pltpu.sync_copy(g_v, out_hbm.at[wid])


_VB = 4096
_NBUF = 3
_GRID = pl.cdiv(_VOCAB, _VB)
_TAILROWS = _VOCAB - (_GRID - 1) * _VB


def _proj_body(w_ref, h_ref, b_ref, out_ref, acc_ref, sem):
    i = pl.program_id(0)
    slot = lax.rem(i, _NBUF)

    @pl.when(i >= _NBUF)
    def _():
        pltpu.make_async_copy(
            acc_ref.at[slot],
            out_ref.at[pl.ds((i - _NBUF) * _VB, _VB), :],
            sem.at[slot],
        ).wait()

    acc = lax.dot_general(
        w_ref[...], h_ref[...],
        (((0,), (0,)), ((), ())),
        preferred_element_type=jnp.float32,
    )
    acc_ref[slot] = acc + jnp.transpose(b_ref[...], (1, 0))

    @pl.when(i < _GRID - 1)
    def _():
        pltpu.make_async_copy(
            acc_ref.at[slot],
            out_ref.at[pl.ds(i * _VB, _VB), :],
            sem.at[slot],
        ).start()

    @pl.when(i == _GRID - 1)
    def _():
        pltpu.make_async_copy(
            acc_ref.at[slot, : _TAILROWS, :],
            out_ref.at[pl.ds((_GRID - 1) * _VB, _TAILROWS), :],
            sem.at[slot],
        ).start()
        for j in range(_GRID - _NBUF, _GRID - 1):
            pltpu.make_async_copy(
                acc_ref.at[j % _NBUF],
                out_ref.at[pl.ds(j * _VB, _VB), :],
                sem.at[j % _NBUF],
            ).wait()
        pltpu.make_async_copy(
            acc_ref.at[(_GRID - 1) % _NBUF, : _TAILROWS, :],
            out_ref.at[pl.ds((_GRID - 1) * _VB, _TAILROWS), :],
            sem.at[(_GRID - 1) % _NBUF],
        ).wait()


_proj = pl.pallas_call(
    _proj_body,
    grid=(_GRID,),
    in_specs=[
        pl.BlockSpec((_HIDDEN, _VB), lambda i: (0, i)),
        pl.BlockSpec((_HIDDEN, _BATCH), lambda i: (0, 0)),
        pl.BlockSpec((1, _VB), lambda i: (0, i)),
    ],
    out_specs=pl.BlockSpec(memory_space=pl.ANY),
    out_shape=jax.ShapeDtypeStruct((_VOCAB, _BATCH), jnp.float32),
    scratch_shapes=[
        pltpu.VMEM((_NBUF, _VB, _BATCH), jnp.float32),
        pltpu.SemaphoreType.DMA((_NBUF,)),
    ],
    compiler_params=pltpu.CompilerParams(vmem_limit_bytes=100 * 1024 * 1024),
)


def kernel(input_ids, embed_table, proj_weight, proj_bias):
    hidden_t = _sc_gather_t(input_ids.astype(jnp.int32), embed_table.T)
    logits_t = _proj(proj_weight.T, hidden_t, proj_bias[None, :])
    return logits_t.T


# speedup vs baseline: 1.0111x; 1.0111x over previous
import functools

import jax
import jax.numpy as jnp
from jax import lax
from jax.experimental import pallas as pl
from jax.experimental.pallas import tpu as pltpu
from jax.experimental.pallas import tpu_sc as plsc

_VOCAB = 100000
_HIDDEN = 32
_BATCH = 1024

_info = plsc.get_sparse_core_info()
_NC, _NS = _info.num_cores, _info.num_subcores
_NW = _NC * _NS
_B_PER_W = _BATCH // _NW

_sc_mesh = plsc.VectorSubcoreMesh(core_axis_name="c", subcore_axis_name="s")


@functools.partial(
    pl.kernel,
    mesh=_sc_mesh,
    out_type=jax.ShapeDtypeStruct((_HIDDEN, _BATCH), jnp.float32),
    scratch_types=[
        pltpu.VMEM((_BATCH,), jnp.int32),
        pltpu.VMEM((_VOCAB,), jnp.float32),
        pltpu.VMEM((_BATCH,), jnp.float32),
    ],
    compiler_params=pltpu.CompilerParams(
        use_tc_tiling_on_sc=True, needs_layout_passes=False),
)
def _sc_gather_t(idx_hbm, table_t_hbm, out_hbm, idx_v, row_v, g_v):
    wid = lax.axis_index("s") * _NC + lax.axis_index("c")
    pltpu.sync_copy(idx_hbm, idx_v)
    pltpu.sync_copy(table_t_hbm.at[wid], row_v)
    for k in range(_BATCH // 16):
        idx16 = idx_v[pl.ds(k * 16, 16)]
        g_v[pl.ds(k * 16, 16)] = plsc.load_gather(row_v, [idx16])
    pltpu.sync_copy(g_v, out_hbm.at[wid])


_VB = 2048
_NBUF = 4
_GRID = pl.cdiv(_VOCAB, _VB)
_TAILROWS = _VOCAB - (_GRID - 1) * _VB


def _proj_body(w_ref, h_ref, b_ref, out_ref, acc_ref, sem):
    i = pl.program_id(0)
    slot = lax.rem(i, _NBUF)

    @pl.when(i >= _NBUF)
    def _():
        pltpu.make_async_copy(
            acc_ref.at[slot],
            out_ref.at[pl.ds((i - _NBUF) * _VB, _VB), :],
            sem.at[slot],
        ).wait()

    acc = lax.dot_general(
        w_ref[...], h_ref[...],
        (((0,), (0,)), ((), ())),
        preferred_element_type=jnp.float32,
    )
    acc_ref[slot] = acc + jnp.transpose(b_ref[...], (1, 0))

    @pl.when(i < _GRID - 1)
    def _():
        pltpu.make_async_copy(
            acc_ref.at[slot],
            out_ref.at[pl.ds(i * _VB, _VB), :],
            sem.at[slot],
        ).start()

    @pl.when(i == _GRID - 1)
    def _():
        pltpu.make_async_copy(
            acc_ref.at[slot, : _TAILROWS, :],
            out_ref.at[pl.ds((_GRID - 1) * _VB, _TAILROWS), :],
            sem.at[slot],
        ).start()
        for j in range(_GRID - _NBUF, _GRID - 1):
            pltpu.make_async_copy(
                acc_ref.at[j % _NBUF],
                out_ref.at[pl.ds(j * _VB, _VB), :],
                sem.at[j % _NBUF],
            ).wait()
        pltpu.make_async_copy(
            acc_ref.at[(_GRID - 1) % _NBUF, : _TAILROWS, :],
            out_ref.at[pl.ds((_GRID - 1) * _VB, _TAILROWS), :],
            sem.at[(_GRID - 1) % _NBUF],
        ).wait()


_proj = pl.pallas_call(
    _proj_body,
    grid=(_GRID,),
    in_specs=[
        pl.BlockSpec((_HIDDEN, _VB), lambda i: (0, i)),
        pl.BlockSpec((_HIDDEN, _BATCH), lambda i: (0, 0)),
        pl.BlockSpec((1, _VB), lambda i: (0, i)),
    ],
    out_specs=pl.BlockSpec(memory_space=pl.ANY),
    out_shape=jax.ShapeDtypeStruct((_VOCAB, _BATCH), jnp.float32),
    scratch_shapes=[
        pltpu.VMEM((_NBUF, _VB, _BATCH), jnp.float32),
        pltpu.SemaphoreType.DMA((_NBUF,)),
    ],
    compiler_params=pltpu.CompilerParams(vmem_limit_bytes=100 * 1024 * 1024),
)


def kernel(input_ids, embed_table, proj_weight, proj_bias):
    hidden_t = _sc_gather_t(input_ids.astype(jnp.int32), embed_table.T)
    logits_t = _proj(proj_weight.T, hidden_t, proj_bias[None, :])
    return logits_t.T
